# baseline (speedup 1.0000x reference)
import functools

import jax
import jax.numpy as jnp
from jax import lax
from jax.experimental import pallas as pl
from jax.experimental.pallas import tpu as pltpu
from jax.experimental.pallas import tpu_sc as plsc

N = 16777216
BINS = 2048
NC = 2
NS = 16
L = 16
NW = NC * NS
PER_W = N // NW
CHUNK = 32768
NCHUNK = PER_W // CHUNK
VPC = CHUNK // L

_sc_mesh = plsc.VectorSubcoreMesh(core_axis_name="c", subcore_axis_name="s")


_U = 16


@functools.partial(
    pl.kernel,
    out_type=(
        jax.ShapeDtypeStruct((NW * L,), jnp.float32),
        jax.ShapeDtypeStruct((NW * L,), jnp.float32),
    ),
    mesh=_sc_mesh,
    compiler_params=pltpu.CompilerParams(needs_layout_passes=False),
    scratch_types=[
        [pltpu.VMEM((CHUNK,), jnp.float32)] * 2,
        pltpu.VMEM((L,), jnp.float32),
        pltpu.VMEM((L,), jnp.float32),
        [pltpu.SemaphoreType.DMA] * 2,
    ],
)
def _sc_minmax(x_hbm, mnp_hbm, mxp_hbm, bufs, mn_v, mx_v, isems):
    c = lax.axis_index("c")
    s = lax.axis_index("s")
    wid = s * NC + c
    base = wid * PER_W

    def chunk_slice(k):
        return x_hbm.at[pl.ds(base + k * CHUNK, CHUNK)]

    in_desc = [
        pltpu.async_copy(chunk_slice(0), bufs[0], isems[0]),
        pltpu.async_copy(chunk_slice(1), bufs[1], isems[1]),
    ]
    in_desc[0].wait()
    waited0 = True

    carry = tuple(bufs[0][pl.ds(u * L, L)] for u in range(_U)) * 2

    for k in range(NCHUNK):
        p = k % 2
        if not (k == 0 and waited0):
            in_desc[p].wait()
        buf = bufs[p]

        @pl.loop(0, VPC // _U, init_carry=carry)
        def _acc(i, carry):
            mns = list(carry[:_U])
            mxs = list(carry[_U:])
            for u in range(_U):
                v = buf[pl.ds((i * _U + u) * L, L)]
                mns[u] = jnp.minimum(mns[u], v)
                mxs[u] = jnp.maximum(mxs[u], v)
            return tuple(mns) + tuple(mxs)

        carry = _acc
        if k + 2 < NCHUNK:
            in_desc[p] = pltpu.async_copy(chunk_slice(k + 2), bufs[p], isems[p])

    mn = carry[0]
    mx = carry[_U]
    for u in range(1, _U):
        mn = jnp.minimum(mn, carry[u])
        mx = jnp.maximum(mx, carry[_U + u])
    mn_v[...] = mn
    mx_v[...] = mx
    pltpu.sync_copy(mn_v, mnp_hbm.at[pl.ds(wid * L, L)])
    pltpu.sync_copy(mx_v, mxp_hbm.at[pl.ds(wid * L, L)])


_NBUF = 3


@functools.partial(
    pl.kernel,
    out_type=(
        jax.ShapeDtypeStruct((NW, BINS + L), jnp.float32),
        jax.ShapeDtypeStruct((N,), jnp.float32),
        jax.ShapeDtypeStruct((L,), jnp.float32),
        jax.ShapeDtypeStruct((L,), jnp.float32),
    ),
    mesh=_sc_mesh,
    compiler_params=pltpu.CompilerParams(needs_layout_passes=False),
    scratch_types=[
        [pltpu.VMEM((CHUNK,), jnp.float32)] * _NBUF,
        pltpu.VMEM((BINS + L,), jnp.float32),
        pltpu.VMEM((NW * L,), jnp.float32),
        pltpu.VMEM((NW * L,), jnp.float32),
        pltpu.VMEM((L,), jnp.float32),
        pltpu.VMEM((L,), jnp.float32),
        [pltpu.SemaphoreType.DMA] * _NBUF,
        [pltpu.SemaphoreType.DMA] * _NBUF,
    ],
)
def _sc_hist(x_hbm, mnp_hbm, mxp_hbm, parts_hbm, xout_hbm, mn_hbm, mx_hbm,
             bufs, hist, mnstage, mxstage, mn_v, mx_v, isems, osems):
    c = lax.axis_index("c")
    s = lax.axis_index("s")
    wid = s * NC + c
    base = wid * PER_W

    pltpu.sync_copy(mnp_hbm, mnstage)
    pltpu.sync_copy(mxp_hbm, mxstage)
    macc = mnstage[pl.ds(0, L)]
    Macc = mxstage[pl.ds(0, L)]
    for w in range(1, NW):
        macc = jnp.minimum(macc, mnstage[pl.ds(w * L, L)])
        Macc = jnp.maximum(Macc, mxstage[pl.ds(w * L, L)])
    mscal = jnp.min(macc)
    Mscal = jnp.max(Macc)
    minv = jnp.broadcast_to(mscal, (L,))
    maxv = jnp.broadcast_to(Mscal, (L,))
    rngv = maxv - minv
    rngv = jnp.where(rngv == 0.0, jnp.float32(1.0), rngv)
    sval = jnp.full((L,), BINS, jnp.float32) / rngv
    ones = jnp.ones((L,), jnp.float32)

    @pl.when(wid == 0)
    def _():
        mn_v[...] = minv
        mx_v[...] = maxv
        pltpu.sync_copy(mn_v, mn_hbm)
        pltpu.sync_copy(mx_v, mx_hbm)

    @pl.loop(0, (BINS + L) // L, unroll=8)
    def _zero(i):
        hist[pl.ds(i * L, L)] = jnp.zeros((L,), jnp.float32)

    def chunk_slice(ref, k):
        return ref.at[pl.ds(base + k * CHUNK, CHUNK)]

    in_desc = [None] * _NBUF
    out_desc = [None] * _NBUF
    for j in range(min(_NBUF - 1, NCHUNK)):
        in_desc[j] = pltpu.async_copy(chunk_slice(x_hbm, j), bufs[j], isems[j])

    for k in range(NCHUNK):
        p = k % _NBUF
        in_desc[p].wait()
        buf = bufs[p]

        @plsc.parallel_loop(0, VPC, unroll=4)
        def _process(i):
            v = buf[pl.ds(i * L, L)]
            t = (v - minv) * sval
            idx = t.astype(jnp.int32)
            plsc.addupdate_scatter(hist, [idx], ones)

        out_desc[p] = pltpu.async_copy(buf, chunk_slice(xout_hbm, k), osems[p])

        j = k + _NBUF - 1
        if j < NCHUNK:
            q = j % _NBUF
            if out_desc[q] is not None:
                out_desc[q].wait()
                out_desc[q] = None
            in_desc[q] = pltpu.async_copy(chunk_slice(x_hbm, j), bufs[q], isems[q])

    for p in range(_NBUF):
        if out_desc[p] is not None:
            out_desc[p].wait()

    pltpu.sync_copy(hist, parts_hbm.at[wid])


def _combine_body(p_ref, h_ref):
    full = jnp.sum(p_ref[...], axis=0, keepdims=True)
    over = full[0, BINS]
    lanes = lax.broadcasted_iota(jnp.int32, (1, BINS), 1)
    h_ref[...] = full[:, :BINS] + jnp.where(
        lanes == BINS - 1, over, jnp.float32(0.0))


def _tc_combine(parts):
    return pl.pallas_call(
        _combine_body,
        out_shape=jax.ShapeDtypeStruct((1, BINS), jnp.float32),
    )(parts)


def kernel(x):
    x_flat = x.reshape(-1)
    mnp, mxp = _sc_minmax(x_flat)
    parts, x_copy, mn16, mx16 = _sc_hist(x_flat, mnp, mxp)
    hist2 = _tc_combine(parts)
    return (x_copy.reshape(x.shape), hist2.reshape(BINS), mn16[0], mx16[0])


# speedup vs baseline: 1.0064x; 1.0064x over previous
import functools

import jax
import jax.numpy as jnp
from jax import lax
from jax.experimental import pallas as pl
from jax.experimental.pallas import tpu as pltpu
from jax.experimental.pallas import tpu_sc as plsc

N = 16777216
BINS = 2048
NC = 2
NS = 16
L = 16
NW = NC * NS
PER_W = N // NW
CHUNK = 32768
NCHUNK = PER_W // CHUNK
VPC = CHUNK // L

_sc_mesh = plsc.VectorSubcoreMesh(core_axis_name="c", subcore_axis_name="s")


_U = 8


@functools.partial(
    pl.kernel,
    out_type=(
        jax.ShapeDtypeStruct((NW * L,), jnp.float32),
        jax.ShapeDtypeStruct((NW * L,), jnp.float32),
    ),
    mesh=_sc_mesh,
    compiler_params=pltpu.CompilerParams(needs_layout_passes=False),
    scratch_types=[
        [pltpu.VMEM((CHUNK,), jnp.float32)] * 2,
        pltpu.VMEM((L,), jnp.float32),
        pltpu.VMEM((L,), jnp.float32),
        [pltpu.SemaphoreType.DMA] * 2,
    ],
)
def _sc_minmax(x_hbm, mnp_hbm, mxp_hbm, bufs, mn_v, mx_v, isems):
    c = lax.axis_index("c")
    s = lax.axis_index("s")
    wid = s * NC + c
    base = wid * PER_W

    def chunk_slice(k):
        return x_hbm.at[pl.ds(base + k * CHUNK, CHUNK)]

    in_desc = [
        pltpu.async_copy(chunk_slice(0), bufs[0], isems[0]),
        pltpu.async_copy(chunk_slice(1), bufs[1], isems[1]),
    ]
    in_desc[0].wait()
    waited0 = True

    carry = tuple(bufs[0][pl.ds(u * L, L)] for u in range(_U)) * 2

    for k in range(NCHUNK):
        p = k % 2
        if not (k == 0 and waited0):
            in_desc[p].wait()
        buf = bufs[p]

        @pl.loop(0, VPC // _U, init_carry=carry)
        def _acc(i, carry):
            mns = list(carry[:_U])
            mxs = list(carry[_U:])
            for u in range(_U):
                v = buf[pl.ds((i * _U + u) * L, L)]
                mns[u] = jnp.minimum(mns[u], v)
                mxs[u] = jnp.maximum(mxs[u], v)
            return tuple(mns) + tuple(mxs)

        carry = _acc
        if k + 2 < NCHUNK:
            in_desc[p] = pltpu.async_copy(chunk_slice(k + 2), bufs[p], isems[p])

    mn = carry[0]
    mx = carry[_U]
    for u in range(1, _U):
        mn = jnp.minimum(mn, carry[u])
        mx = jnp.maximum(mx, carry[_U + u])
    mn_v[...] = mn
    mx_v[...] = mx
    pltpu.sync_copy(mn_v, mnp_hbm.at[pl.ds(wid * L, L)])
    pltpu.sync_copy(mx_v, mxp_hbm.at[pl.ds(wid * L, L)])


_NBUF = 3


@functools.partial(
    pl.kernel,
    out_type=(
        jax.ShapeDtypeStruct((NW, BINS + L), jnp.float32),
        jax.ShapeDtypeStruct((N,), jnp.float32),
        jax.ShapeDtypeStruct((L,), jnp.float32),
        jax.ShapeDtypeStruct((L,), jnp.float32),
    ),
    mesh=_sc_mesh,
    compiler_params=pltpu.CompilerParams(needs_layout_passes=False),
    scratch_types=[
        [pltpu.VMEM((CHUNK,), jnp.float32)] * _NBUF,
        pltpu.VMEM((BINS + L,), jnp.float32),
        pltpu.VMEM((NW * L,), jnp.float32),
        pltpu.VMEM((NW * L,), jnp.float32),
        pltpu.VMEM((L,), jnp.float32),
        pltpu.VMEM((L,), jnp.float32),
        [pltpu.SemaphoreType.DMA] * _NBUF,
        [pltpu.SemaphoreType.DMA] * _NBUF,
    ],
)
def _sc_hist(x_hbm, mnp_hbm, mxp_hbm, parts_hbm, xout_hbm, mn_hbm, mx_hbm,
             bufs, hist, mnstage, mxstage, mn_v, mx_v, isems, osems):
    c = lax.axis_index("c")
    s = lax.axis_index("s")
    wid = s * NC + c
    base = wid * PER_W

    pltpu.sync_copy(mnp_hbm, mnstage)
    pltpu.sync_copy(mxp_hbm, mxstage)
    macc = mnstage[pl.ds(0, L)]
    Macc = mxstage[pl.ds(0, L)]
    for w in range(1, NW):
        macc = jnp.minimum(macc, mnstage[pl.ds(w * L, L)])
        Macc = jnp.maximum(Macc, mxstage[pl.ds(w * L, L)])
    mscal = jnp.min(macc)
    Mscal = jnp.max(Macc)
    minv = jnp.broadcast_to(mscal, (L,))
    maxv = jnp.broadcast_to(Mscal, (L,))
    rngv = maxv - minv
    rngv = jnp.where(rngv == 0.0, jnp.float32(1.0), rngv)
    sval = jnp.full((L,), BINS, jnp.float32) / rngv
    sval = jnp.where(sval == jnp.float32(jnp.inf), jnp.float32(0.0), sval)
    ones = jnp.ones((L,), jnp.float32)

    @pl.when(wid == 0)
    def _():
        mn_v[...] = minv
        mx_v[...] = maxv
        pltpu.sync_copy(mn_v, mn_hbm)
        pltpu.sync_copy(mx_v, mx_hbm)

    @pl.loop(0, (BINS + L) // L, unroll=8)
    def _zero(i):
        hist[pl.ds(i * L, L)] = jnp.zeros((L,), jnp.float32)

    def chunk_slice(ref, k):
        return ref.at[pl.ds(base + k * CHUNK, CHUNK)]

    in_desc = [None] * _NBUF
    out_desc = [None] * _NBUF
    for j in range(min(_NBUF - 1, NCHUNK)):
        in_desc[j] = pltpu.async_copy(chunk_slice(x_hbm, j), bufs[j], isems[j])

    for k in range(NCHUNK):
        p = k % _NBUF
        in_desc[p].wait()
        buf = bufs[p]

        @plsc.parallel_loop(0, VPC, unroll=4)
        def _process(i):
            v = buf[pl.ds(i * L, L)]
            t = (v - minv) * sval
            idx = t.astype(jnp.int32)
            plsc.addupdate_scatter(hist, [idx], ones)

        out_desc[p] = pltpu.async_copy(buf, chunk_slice(xout_hbm, k), osems[p])

        j = k + _NBUF - 1
        if j < NCHUNK:
            q = j % _NBUF
            if out_desc[q] is not None:
                out_desc[q].wait()
                out_desc[q] = None
            in_desc[q] = pltpu.async_copy(chunk_slice(x_hbm, j), bufs[q], isems[q])

    for p in range(_NBUF):
        if out_desc[p] is not None:
            out_desc[p].wait()

    pltpu.sync_copy(hist, parts_hbm.at[wid])


def _combine_body(p_ref, h_ref):
    full = jnp.sum(p_ref[...], axis=0, keepdims=True)
    over = full[0, BINS]
    lanes = lax.broadcasted_iota(jnp.int32, (1, BINS), 1)
    h_ref[...] = full[:, :BINS] + jnp.where(
        lanes == BINS - 1, over, jnp.float32(0.0))


def _tc_combine(parts):
    return pl.pallas_call(
        _combine_body,
        out_shape=jax.ShapeDtypeStruct((1, BINS), jnp.float32),
    )(parts)


def kernel(x):
    x_flat = x.reshape(-1)
    mnp, mxp = _sc_minmax(x_flat)
    parts, x_copy, mn16, mx16 = _sc_hist(x_flat, mnp, mxp)
    hist2 = _tc_combine(parts)
    return (x_copy.reshape(x.shape), hist2.reshape(BINS), mn16[0], mx16[0])
